# Initial kernel scaffold; baseline (speedup 1.0000x reference)
import functools

import jax
import jax.numpy as jnp
from jax import lax
from jax.experimental import pallas as pl
from jax.experimental.pallas import tpu as pltpu
from jax.experimental.pallas import tpu_sc as plsc

B, T, D = 4096, 200, 64
N = B * T
NC, NS = 2, 16
NW = NC * NS
CHUNK = N // NW
K = 256
STEPS = CHUNK // K
LANES = 16


@functools.partial(
    pl.kernel,
    out_type=jax.ShapeDtypeStruct((N, D), jnp.float32),
    mesh=plsc.VectorSubcoreMesh(core_axis_name="c", subcore_axis_name="s"),
    scratch_types=[
        pltpu.VMEM((K,), jnp.int32),
        pltpu.VMEM((K,), jnp.int32),
        pltpu.VMEM((K,), jnp.int32),
        pltpu.VMEM((K, D), jnp.float32),
        pltpu.VMEM((K, D), jnp.float32),
        pltpu.VMEM((K, D), jnp.float32),
        pltpu.SemaphoreType.DMA,
    ],
)
def _encode(i1, i2, i3, t1, t2, t3, out, x1, x2, x3, r1, r2, r3, sem):
    wid = lax.axis_index("s") * NC + lax.axis_index("c")
    base = wid * CHUNK

    def step(g, carry):
        off = base + g * K
        pltpu.sync_copy(i1.at[pl.ds(off, K)], x1)
        pltpu.sync_copy(i2.at[pl.ds(off, K)], x2)
        pltpu.sync_copy(i3.at[pl.ds(off, K)], x3)
        cp1 = pltpu.async_copy(t1.at[x1], r1, sem)
        cp2 = pltpu.async_copy(t2.at[x2], r2, sem)
        cp3 = pltpu.async_copy(t3.at[x3], r3, sem)
        cp1.wait()
        cp2.wait()
        cp3.wait()

        def add_row(i, c):
            for j in range(D // LANES):
                sl = pl.ds(j * LANES, LANES)
                r1[i, sl] = r1[i, sl] + r2[i, sl] + r3[i, sl]
            return c

        lax.fori_loop(0, K, add_row, 0)
        pltpu.sync_copy(r1, out.at[pl.ds(off, K)])
        return carry

    lax.fori_loop(0, STEPS, step, 0)


def kernel(mcc_code, tr_type, country, seq_lens, emb_mcc, emb_tr, emb_cty):
    t1 = emb_mcc.at[0].set(0.0)
    t2 = emb_tr.at[0].set(0.0)
    t3 = emb_cty.at[0].set(0.0)
    i1 = mcc_code.reshape(N)
    i2 = tr_type.reshape(N)
    i3 = country.reshape(N)
    out = _encode(i1, i2, i3, t1, t2, t3)
    return out.reshape(B, T, D)


# speedup vs baseline: 6.5992x; 6.5992x over previous
import functools

import jax
import jax.numpy as jnp
from jax import lax
from jax.experimental import pallas as pl
from jax.experimental.pallas import tpu as pltpu
from jax.experimental.pallas import tpu_sc as plsc

B, T, D = 4096, 200, 64
N = B * T
NC, NS = 2, 16
NW = NC * NS
CHUNK = N // NW
K = 256
STEPS = CHUNK // K
LANES = 16


@functools.partial(
    pl.kernel,
    out_type=jax.ShapeDtypeStruct((N, D), jnp.float32),
    mesh=plsc.VectorSubcoreMesh(core_axis_name="c", subcore_axis_name="s"),
    scratch_types=[
        pltpu.VMEM((K,), jnp.int32),
        pltpu.VMEM((K,), jnp.int32),
        pltpu.VMEM((K,), jnp.int32),
        pltpu.VMEM((K, D), jnp.float32),
        pltpu.VMEM((K, D), jnp.float32),
        pltpu.VMEM((K, D), jnp.float32),
        pltpu.SemaphoreType.DMA,
    ],
    compiler_params=pltpu.CompilerParams(use_tc_tiling_on_sc=False),
)
def _encode(i1, i2, i3, t1, t2, t3, out, x1, x2, x3, r1, r2, r3, sem):
    wid = lax.axis_index("s") * NC + lax.axis_index("c")
    base = wid * CHUNK

    def step(g, carry):
        off = base + g * K
        pltpu.sync_copy(i1.at[pl.ds(off, K)], x1)
        pltpu.sync_copy(i2.at[pl.ds(off, K)], x2)
        pltpu.sync_copy(i3.at[pl.ds(off, K)], x3)
        cp1 = pltpu.async_copy(t1.at[x1], r1, sem)
        cp2 = pltpu.async_copy(t2.at[x2], r2, sem)
        cp3 = pltpu.async_copy(t3.at[x3], r3, sem)
        cp1.wait()
        cp2.wait()
        cp3.wait()

        def add_row(i, c):
            for j in range(D // LANES):
                sl = pl.ds(j * LANES, LANES)
                r1[i, sl] = r1[i, sl] + r2[i, sl] + r3[i, sl]
            return c

        lax.fori_loop(0, K, add_row, 0)
        pltpu.sync_copy(r1, out.at[pl.ds(off, K)])
        return carry

    lax.fori_loop(0, STEPS, step, 0)


def kernel(mcc_code, tr_type, country, seq_lens, emb_mcc, emb_tr, emb_cty):
    t1 = emb_mcc.at[0].set(0.0)
    t2 = emb_tr.at[0].set(0.0)
    t3 = emb_cty.at[0].set(0.0)
    i1 = mcc_code.reshape(N)
    i2 = tr_type.reshape(N)
    i3 = country.reshape(N)
    out = _encode(i1, i2, i3, t1, t2, t3)
    return out.reshape(B, T, D)


# speedup vs baseline: 8.6853x; 1.3161x over previous
import functools

import jax
import jax.numpy as jnp
from jax import lax
from jax.experimental import pallas as pl
from jax.experimental.pallas import tpu as pltpu
from jax.experimental.pallas import tpu_sc as plsc

B, T, D = 4096, 200, 64
N = B * T
NC, NS = 2, 16
NW = NC * NS
CHUNK = N // NW
K = 128
STEPS = CHUNK // K
NPAIR = STEPS // 2
LANES = 16
ROW_BYTES = K * D * 4


@functools.partial(
    pl.kernel,
    out_type=jax.ShapeDtypeStruct((N, D), jnp.float32),
    mesh=plsc.VectorSubcoreMesh(core_axis_name="c", subcore_axis_name="s"),
    scratch_types=[
        pltpu.VMEM((CHUNK,), jnp.int32),
        pltpu.VMEM((CHUNK,), jnp.int32),
        pltpu.VMEM((CHUNK,), jnp.int32),
        pltpu.VMEM((K, D), jnp.float32),
        pltpu.VMEM((K, D), jnp.float32),
        pltpu.VMEM((K, D), jnp.float32),
        pltpu.VMEM((K, D), jnp.float32),
        pltpu.VMEM((K, D), jnp.float32),
        pltpu.VMEM((K, D), jnp.float32),
        pltpu.SemaphoreType.DMA,
        pltpu.SemaphoreType.DMA,
        pltpu.SemaphoreType.DMA,
        pltpu.SemaphoreType.DMA,
    ],
    compiler_params=pltpu.CompilerParams(use_tc_tiling_on_sc=False),
)
def _encode(i1, i2, i3, t1, t2, t3, out,
            x1, x2, x3, a1, a2, a3, b1, b2, b3, gsa, gsb, osa, osb):
    wid = lax.axis_index("s") * NC + lax.axis_index("c")
    base = wid * CHUNK
    pltpu.sync_copy(i1.at[pl.ds(base, CHUNK)], x1)
    pltpu.sync_copy(i2.at[pl.ds(base, CHUNK)], x2)
    pltpu.sync_copy(i3.at[pl.ds(base, CHUNK)], x3)

    def gather(s, d1, d2, d3, sem):
        o = pl.multiple_of(s * K, K)
        pltpu.async_copy(t1.at[x1.at[pl.ds(o, K)]], d1, sem)
        pltpu.async_copy(t2.at[x2.at[pl.ds(o, K)]], d2, sem)
        pltpu.async_copy(t3.at[x3.at[pl.ds(o, K)]], d3, sem)

    def wait_gathers(d1, d2, d3, sem):
        pltpu.make_async_copy(t1.at[x1.at[pl.ds(0, K)]], d1, sem).wait()
        pltpu.make_async_copy(t2.at[x2.at[pl.ds(0, K)]], d2, sem).wait()
        pltpu.make_async_copy(t3.at[x3.at[pl.ds(0, K)]], d3, sem).wait()

    def start_write(s, d1, sem):
        o = pl.multiple_of(base + s * K, K)
        pltpu.async_copy(d1, out.at[pl.ds(o, K)], sem)

    def wait_write(d1, sem):
        pltpu.make_async_copy(d1, out.at[pl.ds(0, K)], sem).wait()

    def add_set(d1, d2, d3):
        def body(i, c):
            r = i * 4
            for rr in range(4):
                for j in range(D // LANES):
                    sl = pl.ds(j * LANES, LANES)
                    d1[r + rr, sl] = d1[r + rr, sl] + d2[r + rr, sl] + d3[r + rr, sl]
            return c
        lax.fori_loop(0, K // 4, body, 0)

    gather(0, a1, a2, a3, gsa)

    def pair(gg, c):
        g0 = gg * 2

        @pl.when(gg > 0)
        def _():
            wait_write(b1, osb)
        gather(g0 + 1, b1, b2, b3, gsb)
        wait_gathers(a1, a2, a3, gsa)
        add_set(a1, a2, a3)
        start_write(g0, a1, osa)

        @pl.when(gg < NPAIR - 1)
        def _():
            wait_write(a1, osa)
            gather(g0 + 2, a1, a2, a3, gsa)
        wait_gathers(b1, b2, b3, gsb)
        add_set(b1, b2, b3)
        start_write(g0 + 1, b1, osb)
        return c

    lax.fori_loop(0, NPAIR, pair, 0)
    wait_write(a1, osa)
    wait_write(b1, osb)


def kernel(mcc_code, tr_type, country, seq_lens, emb_mcc, emb_tr, emb_cty):
    t1 = emb_mcc.at[0].set(0.0)
    t2 = emb_tr.at[0].set(0.0)
    t3 = emb_cty.at[0].set(0.0)
    i1 = mcc_code.reshape(N)
    i2 = tr_type.reshape(N)
    i3 = country.reshape(N)
    out = _encode(i1, i2, i3, t1, t2, t3)
    return out.reshape(B, T, D)
